# Initial kernel scaffold; baseline (speedup 1.0000x reference)
import jax
import jax.numpy as jnp
from jax.experimental import pallas as pl


def kernel(x, emb_w, emb_b, bn_g, bn_b, sg_w, sg_b, l1_w, l1_b, l2_w, l2_b, l3_w, l3_b, edge_index):
    raise NotImplementedError("write your pallas kernel here")


# speedup vs baseline: 14.9532x; 14.9532x over previous
import functools

import numpy as np
import jax
import jax.numpy as jnp
from jax import lax
from jax.experimental import pallas as pl
from jax.experimental.pallas import tpu as pltpu
from jax.experimental.pallas import tpu_sc as plsc

_N = 10000
_E = 320000
_H = 128
_NP = 10240
_NW = 32
_EW = _E // _NW
_K = 80
_CH = _EW // _K
_TPS = _NP // 16
_DD = 128

_mesh = plsc.VectorSubcoreMesh(core_axis_name="c", subcore_axis_name="s")


@functools.partial(
    pl.kernel,
    out_type=jax.ShapeDtypeStruct((2, _NP, _DD), jnp.float32),
    mesh=_mesh,
    scratch_types=[
        pltpu.VMEM((_K,), jnp.int32),
        pltpu.VMEM((_K, _DD), jnp.float32),
        pltpu.VMEM_SHARED((_NP, _DD), jnp.float32),
    ],
)
def _sc_degree(col_hbm, zeros_hbm, ones_hbm, out_hbm, colv, onesv, acc):
    cid = lax.axis_index("c")
    sid = lax.axis_index("s")
    wid = sid * 2 + cid
    pltpu.sync_copy(zeros_hbm, acc.at[pl.ds(sid * _TPS, _TPS)])
    pltpu.sync_copy(ones_hbm, onesv)
    plsc.subcore_barrier()

    def body(j, carry):
        base = wid * _EW + j * _K
        pltpu.sync_copy(col_hbm.at[pl.ds(base, _K)], colv)
        pltpu.sync_copy(onesv, acc.at[colv], add=True)
        return carry

    lax.fori_loop(0, _CH, body, 0)
    plsc.subcore_barrier()
    pltpu.sync_copy(acc.at[pl.ds(sid * _TPS, _TPS)],
                    out_hbm.at[cid, pl.ds(sid * _TPS, _TPS)])


@functools.partial(
    pl.kernel,
    out_type=jax.ShapeDtypeStruct((2, _NP, _H), jnp.float32),
    mesh=_mesh,
    scratch_types=[
        pltpu.VMEM((_K,), jnp.int32),
        pltpu.VMEM((_K,), jnp.int32),
        pltpu.VMEM((_K, _H), jnp.float32),
        pltpu.VMEM_SHARED((_NP, _H), jnp.float32),
        pltpu.SemaphoreType.DMA,
    ],
)
def _sc_scatter(g_hbm, row_hbm, col_hbm, zeros_hbm, out_hbm,
                rowv, colv, rbuf, acc, sem):
    cid = lax.axis_index("c")
    sid = lax.axis_index("s")
    wid = sid * 2 + cid
    for t in range(_TPS // 128):
        pltpu.sync_copy(zeros_hbm, acc.at[pl.ds(sid * _TPS + t * 128, 128)])
    plsc.subcore_barrier()

    def body(j, carry):
        base = wid * _EW + j * _K
        pltpu.sync_copy(row_hbm.at[pl.ds(base, _K)], rowv)
        pltpu.sync_copy(col_hbm.at[pl.ds(base, _K)], colv)
        pltpu.async_copy(g_hbm.at[rowv], rbuf, sem).wait()
        pltpu.sync_copy(rbuf, acc.at[colv], add=True)
        return carry

    lax.fori_loop(0, _CH, body, 0)
    plsc.subcore_barrier()
    pltpu.sync_copy(acc.at[pl.ds(sid * _TPS, _TPS)],
                    out_hbm.at[cid, pl.ds(sid * _TPS, _TPS)])


def _tc_embed_body(x_ref, w_ref, b_ref, bng_ref, bnb_ref, deg_ref,
                   g_out, dis_out):
    h = jnp.dot(x_ref[...], w_ref[...], preferred_element_type=jnp.float32)
    h = jnp.maximum(h + b_ref[...], 0.0)
    scale = bng_ref[...] * np.float32(1.0 / np.sqrt(1.0 + 1e-5))
    h = h * scale + bnb_ref[...]
    dis = lax.rsqrt(deg_ref[...] + 1.0)
    dis_out[...] = dis
    g_out[...] = h * dis


def _tc_head_body(sp_ref, g_ref, dis_ref, sgw_ref, sgb_ref, w1_ref, b1_ref,
                  w2_ref, b2_ref, w3_ref, b3_ref, out_ref):
    s = sp_ref[0] + sp_ref[1] + g_ref[...]
    agg = s * dis_ref[...]
    h = jnp.dot(agg, sgw_ref[...], preferred_element_type=jnp.float32)
    h = jnp.maximum(h + sgb_ref[...], 0.0)
    h = jnp.dot(h, w1_ref[...], preferred_element_type=jnp.float32)
    h = jnp.maximum(h + b1_ref[...], 0.0)
    h = jnp.dot(h, w2_ref[...], preferred_element_type=jnp.float32)
    h = jnp.maximum(h + b2_ref[...], 0.0)
    z = jnp.dot(h, w3_ref[...], preferred_element_type=jnp.float32)
    z = z + b3_ref[...]
    out_ref[...] = 1.0 / (1.0 + jnp.exp(-z))


_B1 = 1000
_G1 = _N // _B1


def kernel(x, emb_w, emb_b, bn_g, bn_b, sg_w, sg_b, l1_w, l1_b,
           l2_w, l2_b, l3_w, l3_b, edge_index):
    row = edge_index[0]
    col = edge_index[1]

    degp = _sc_degree(col,
                      jnp.zeros((_TPS, _DD), jnp.float32),
                      jnp.ones((_K, _DD), jnp.float32))
    deg = (degp[0, :_N, 0] + degp[1, :_N, 0]).reshape(_N, 1)

    g, dis = pl.pallas_call(
        _tc_embed_body,
        grid=(_G1,),
        in_specs=[
            pl.BlockSpec((_B1, _H), lambda i: (i, 0)),
            pl.BlockSpec((_H, _H), lambda i: (0, 0)),
            pl.BlockSpec((1, _H), lambda i: (0, 0)),
            pl.BlockSpec((1, _H), lambda i: (0, 0)),
            pl.BlockSpec((1, _H), lambda i: (0, 0)),
            pl.BlockSpec((_B1, 1), lambda i: (i, 0)),
        ],
        out_specs=[
            pl.BlockSpec((_B1, _H), lambda i: (i, 0)),
            pl.BlockSpec((_B1, 1), lambda i: (i, 0)),
        ],
        out_shape=[
            jax.ShapeDtypeStruct((_N, _H), jnp.float32),
            jax.ShapeDtypeStruct((_N, 1), jnp.float32),
        ],
    )(x, emb_w, emb_b.reshape(1, _H), bn_g.reshape(1, _H),
      bn_b.reshape(1, _H), deg)

    sp = _sc_scatter(g, row, col, jnp.zeros((128, _H), jnp.float32))
    sp = sp[:, :_N, :]

    out = pl.pallas_call(
        _tc_head_body,
        grid=(_G1,),
        in_specs=[
            pl.BlockSpec((2, _B1, _H), lambda i: (0, i, 0)),
            pl.BlockSpec((_B1, _H), lambda i: (i, 0)),
            pl.BlockSpec((_B1, 1), lambda i: (i, 0)),
            pl.BlockSpec((_H, _H), lambda i: (0, 0)),
            pl.BlockSpec((1, _H), lambda i: (0, 0)),
            pl.BlockSpec((_H, 64), lambda i: (0, 0)),
            pl.BlockSpec((1, 64), lambda i: (0, 0)),
            pl.BlockSpec((64, 32), lambda i: (0, 0)),
            pl.BlockSpec((1, 32), lambda i: (0, 0)),
            pl.BlockSpec((32, 18), lambda i: (0, 0)),
            pl.BlockSpec((1, 18), lambda i: (0, 0)),
        ],
        out_specs=pl.BlockSpec((_B1, 18), lambda i: (i, 0)),
        out_shape=jax.ShapeDtypeStruct((_N, 18), jnp.float32),
    )(sp, g, dis, sg_w, sg_b.reshape(1, _H), l1_w, l1_b.reshape(1, 64),
      l2_w, l2_b.reshape(1, 32), l3_w, l3_b.reshape(1, 18))
    return out


# speedup vs baseline: 28.0532x; 1.8761x over previous
import functools

import numpy as np
import jax
import jax.numpy as jnp
from jax import lax
from jax.experimental import pallas as pl
from jax.experimental.pallas import tpu as pltpu
from jax.experimental.pallas import tpu_sc as plsc

_N = 10000
_E = 320000
_H = 128
_NP = 10240
_NW = 32
_EW = _E // _NW
_K = 80
_CH = _EW // _K
_TPS = _NP // 16
_DD = 128

_mesh = plsc.VectorSubcoreMesh(core_axis_name="c", subcore_axis_name="s")


@functools.partial(
    pl.kernel,
    out_type=jax.ShapeDtypeStruct((2, _NP, _DD), jnp.float32),
    mesh=_mesh,
    scratch_types=[
        pltpu.VMEM((_K,), jnp.int32),
        pltpu.VMEM((_K,), jnp.int32),
        pltpu.VMEM((_K, _DD), jnp.float32),
        pltpu.VMEM_SHARED((_NP, _DD), jnp.float32),
        pltpu.SemaphoreType.DMA,
        pltpu.SemaphoreType.DMA,
    ],
)
def _sc_degree(col_hbm, zeros_hbm, ones_hbm, out_hbm, colv0, colv1,
               onesv, acc, semC, semD):
    cid = lax.axis_index("c")
    sid = lax.axis_index("s")
    wid = sid * 2 + cid
    pltpu.sync_copy(zeros_hbm, acc.at[pl.ds(sid * _TPS, _TPS)])
    pltpu.sync_copy(ones_hbm, onesv)
    plsc.subcore_barrier()

    def cp_col(j, dst, sem):
        pltpu.async_copy(col_hbm.at[pl.ds(wid * _EW + j * _K, _K)], dst, sem)

    def cwait(j, dst, sem):
        pltpu.make_async_copy(
            col_hbm.at[pl.ds(wid * _EW + j * _K, _K)], dst, sem).wait()

    cp_col(0, colv0, semC)
    cp_col(1, colv1, semD)

    def body(i, carry):
        j = 2 * i
        cwait(j, colv0, semC)
        pltpu.sync_copy(onesv, acc.at[colv0], add=True)
        cp_col(j + 2, colv0, semC)
        cwait(j + 1, colv1, semD)
        pltpu.sync_copy(onesv, acc.at[colv1], add=True)
        cp_col(j + 3, colv1, semD)
        return carry

    lax.fori_loop(0, (_CH - 1) // 2, body, 0)
    cwait(_CH - 1, colv0, semC)
    pltpu.sync_copy(onesv, acc.at[colv0], add=True)
    cwait(_CH, colv1, semD)
    plsc.subcore_barrier()
    pltpu.sync_copy(acc.at[pl.ds(sid * _TPS, _TPS)],
                    out_hbm.at[cid, pl.ds(sid * _TPS, _TPS)])


@functools.partial(
    pl.kernel,
    out_type=jax.ShapeDtypeStruct((2, _NP, _H), jnp.float32),
    mesh=_mesh,
    scratch_types=[
        pltpu.VMEM((_EW,), jnp.int32),
        pltpu.VMEM((_K,), jnp.int32),
        pltpu.VMEM((_K,), jnp.int32),
        pltpu.VMEM((_K, _H), jnp.float32),
        pltpu.VMEM((_K, _H), jnp.float32),
        pltpu.VMEM_SHARED((_NP, _H), jnp.float32),
        pltpu.SemaphoreType.DMA,
        pltpu.SemaphoreType.DMA,
        pltpu.SemaphoreType.DMA,
        pltpu.SemaphoreType.DMA,
    ],
)
def _sc_scatter(g_hbm, row_hbm, col_hbm, zeros_hbm, out_hbm,
                rows, colv0, colv1, buf0, buf1, acc,
                semA, semB, semC, semD):
    cid = lax.axis_index("c")
    sid = lax.axis_index("s")
    wid = sid * 2 + cid
    pltpu.sync_copy(row_hbm.at[pl.ds(wid * _EW, _EW)], rows)
    for t in range(_TPS // 128):
        pltpu.sync_copy(zeros_hbm, acc.at[pl.ds(sid * _TPS + t * 128, 128)])
    plsc.subcore_barrier()

    def gidx(j):
        return rows.at[pl.ds(j * _K, _K)]

    def cp_col(j, dst, sem):
        pltpu.async_copy(col_hbm.at[pl.ds(wid * _EW + j * _K, _K)], dst, sem)

    def cwait(j, dst, sem):
        pltpu.make_async_copy(
            col_hbm.at[pl.ds(wid * _EW + j * _K, _K)], dst, sem).wait()

    cp_col(0, colv0, semC)
    cp_col(1, colv1, semD)
    pltpu.async_copy(g_hbm.at[gidx(0)], buf0, semA)

    def body(i, carry):
        j = 2 * i
        pltpu.async_copy(g_hbm.at[gidx(j + 1)], buf1, semB)
        pltpu.make_async_copy(g_hbm.at[gidx(j)], buf0, semA).wait()
        cwait(j, colv0, semC)
        pltpu.sync_copy(buf0, acc.at[colv0], add=True)
        cp_col(j + 2, colv0, semC)
        pltpu.async_copy(g_hbm.at[gidx(j + 2)], buf0, semA)
        pltpu.make_async_copy(g_hbm.at[gidx(j + 1)], buf1, semB).wait()
        cwait(j + 1, colv1, semD)
        pltpu.sync_copy(buf1, acc.at[colv1], add=True)
        cp_col(j + 3, colv1, semD)
        return carry

    lax.fori_loop(0, (_CH - 1) // 2, body, 0)
    pltpu.make_async_copy(g_hbm.at[gidx(_CH - 1)], buf0, semA).wait()
    cwait(_CH - 1, colv0, semC)
    pltpu.sync_copy(buf0, acc.at[colv0], add=True)
    cwait(_CH, colv1, semD)
    plsc.subcore_barrier()
    pltpu.sync_copy(acc.at[pl.ds(sid * _TPS, _TPS)],
                    out_hbm.at[cid, pl.ds(sid * _TPS, _TPS)])


def _tc_embed_body(x_ref, w_ref, b_ref, bng_ref, bnb_ref, deg_ref,
                   g_out, dis_out):
    h = jnp.dot(x_ref[...], w_ref[...], preferred_element_type=jnp.float32)
    h = jnp.maximum(h + b_ref[...], 0.0)
    scale = bng_ref[...] * np.float32(1.0 / np.sqrt(1.0 + 1e-5))
    h = h * scale + bnb_ref[...]
    dis = lax.rsqrt(deg_ref[...] + 1.0)
    dis_out[...] = dis
    g_out[...] = h * dis


def _tc_head_body(sp_ref, g_ref, dis_ref, sgw_ref, sgb_ref, w1_ref, b1_ref,
                  w2_ref, b2_ref, w3_ref, b3_ref, out_ref):
    s = sp_ref[0] + sp_ref[1] + g_ref[...]
    agg = s * dis_ref[...]
    h = jnp.dot(agg, sgw_ref[...], preferred_element_type=jnp.float32)
    h = jnp.maximum(h + sgb_ref[...], 0.0)
    h = jnp.dot(h, w1_ref[...], preferred_element_type=jnp.float32)
    h = jnp.maximum(h + b1_ref[...], 0.0)
    h = jnp.dot(h, w2_ref[...], preferred_element_type=jnp.float32)
    h = jnp.maximum(h + b2_ref[...], 0.0)
    z = jnp.dot(h, w3_ref[...], preferred_element_type=jnp.float32)
    z = z + b3_ref[...]
    out_ref[...] = 1.0 / (1.0 + jnp.exp(-z))


_B1 = 1000
_G1 = _N // _B1


def kernel(x, emb_w, emb_b, bn_g, bn_b, sg_w, sg_b, l1_w, l1_b,
           l2_w, l2_b, l3_w, l3_b, edge_index):
    row = edge_index[0]
    col = jnp.concatenate([edge_index[1], jnp.zeros((2 * _K,), jnp.int32)])

    degp = _sc_degree(col,
                      jnp.zeros((_TPS, _DD), jnp.float32),
                      jnp.ones((_K, _DD), jnp.float32))
    deg = (degp[0, :_N, 0] + degp[1, :_N, 0]).reshape(_N, 1)

    g, dis = pl.pallas_call(
        _tc_embed_body,
        grid=(_G1,),
        in_specs=[
            pl.BlockSpec((_B1, _H), lambda i: (i, 0)),
            pl.BlockSpec((_H, _H), lambda i: (0, 0)),
            pl.BlockSpec((1, _H), lambda i: (0, 0)),
            pl.BlockSpec((1, _H), lambda i: (0, 0)),
            pl.BlockSpec((1, _H), lambda i: (0, 0)),
            pl.BlockSpec((_B1, 1), lambda i: (i, 0)),
        ],
        out_specs=[
            pl.BlockSpec((_B1, _H), lambda i: (i, 0)),
            pl.BlockSpec((_B1, 1), lambda i: (i, 0)),
        ],
        out_shape=[
            jax.ShapeDtypeStruct((_N, _H), jnp.float32),
            jax.ShapeDtypeStruct((_N, 1), jnp.float32),
        ],
    )(x, emb_w, emb_b.reshape(1, _H), bn_g.reshape(1, _H),
      bn_b.reshape(1, _H), deg)

    sp = _sc_scatter(g, row, col, jnp.zeros((128, _H), jnp.float32))
    sp = sp[:, :_N, :]

    out = pl.pallas_call(
        _tc_head_body,
        grid=(_G1,),
        in_specs=[
            pl.BlockSpec((2, _B1, _H), lambda i: (0, i, 0)),
            pl.BlockSpec((_B1, _H), lambda i: (i, 0)),
            pl.BlockSpec((_B1, 1), lambda i: (i, 0)),
            pl.BlockSpec((_H, _H), lambda i: (0, 0)),
            pl.BlockSpec((1, _H), lambda i: (0, 0)),
            pl.BlockSpec((_H, 64), lambda i: (0, 0)),
            pl.BlockSpec((1, 64), lambda i: (0, 0)),
            pl.BlockSpec((64, 32), lambda i: (0, 0)),
            pl.BlockSpec((1, 32), lambda i: (0, 0)),
            pl.BlockSpec((32, 18), lambda i: (0, 0)),
            pl.BlockSpec((1, 18), lambda i: (0, 0)),
        ],
        out_specs=pl.BlockSpec((_B1, 18), lambda i: (i, 0)),
        out_shape=jax.ShapeDtypeStruct((_N, 18), jnp.float32),
    )(sp, g, dis, sg_w, sg_b.reshape(1, _H), l1_w, l1_b.reshape(1, 64),
      l2_w, l2_b.reshape(1, 32), l3_w, l3_b.reshape(1, 18))
    return out


# speedup vs baseline: 30.6492x; 1.0925x over previous
import functools

import numpy as np
import jax
import jax.numpy as jnp
from jax import lax
from jax.experimental import pallas as pl
from jax.experimental.pallas import tpu as pltpu
from jax.experimental.pallas import tpu_sc as plsc

_N = 10000
_E = 320000
_H = 128
_NP = 10240
_NW = 32
_EW = _E // _NW
_K = 80
_CH = _EW // _K
_TPS = _NP // 16
_DD = 128

_mesh = plsc.VectorSubcoreMesh(core_axis_name="c", subcore_axis_name="s")


@functools.partial(
    pl.kernel,
    out_type=jax.ShapeDtypeStruct((2, _NP, _DD), jnp.float32),
    mesh=_mesh,
    scratch_types=[
        [pltpu.VMEM((_K,), jnp.int32)] * 8,
        pltpu.VMEM((_K, _DD), jnp.float32),
        pltpu.VMEM_SHARED((_NP, _DD), jnp.float32),
        pltpu.SemaphoreType.DMA((8,)),
        pltpu.SemaphoreType.DMA((4,)),
    ],
)
def _sc_degree(col_hbm, zeros_hbm, ones_hbm, out_hbm, colv,
               onesv, acc, semC, semS):
    cid = lax.axis_index("c")
    sid = lax.axis_index("s")
    wid = sid * 2 + cid
    pltpu.sync_copy(zeros_hbm, acc.at[pl.ds(sid * _TPS, _TPS)])
    pltpu.sync_copy(ones_hbm, onesv)
    plsc.subcore_barrier()

    def csl(j):
        return col_hbm.at[pl.ds(wid * _EW + j * _K, _K)]

    def step(j, u, wait_scatter=True):
        s8, s4, a8 = u % 8, u % 4, (u + 4) % 8
        if wait_scatter:
            pltpu.make_async_copy(onesv, acc.at[colv[a8]], semS.at[s4]).wait()
        pltpu.async_copy(csl(j + 4), colv[a8], semC.at[a8])
        pltpu.make_async_copy(csl(j), colv[s8], semC.at[s8]).wait()
        pltpu.async_copy(onesv, acc.at[colv[s8]], semS.at[s4], add=True)

    for u in range(4):
        pltpu.async_copy(csl(u), colv[u], semC.at[u])
    for u in range(8):
        step(u, u, wait_scatter=(u >= 4))

    def body(i, carry):
        for u in range(8):
            step(8 * i + u, u)
        return carry

    lax.fori_loop(1, 15, body, 0)
    for u in range(5):
        step(120 + u, u)
    for j, u in ((121, 1), (122, 2), (123, 3), (124, 0)):
        pltpu.make_async_copy(onesv, acc.at[colv[u % 8]], semS.at[u % 4]).wait()
    for j, a in ((125, 5), (126, 6), (127, 7), (128, 0)):
        pltpu.make_async_copy(csl(j), colv[a], semC.at[a]).wait()
    plsc.subcore_barrier()
    pltpu.sync_copy(acc.at[pl.ds(sid * _TPS, _TPS)],
                    out_hbm.at[cid, pl.ds(sid * _TPS, _TPS)])


@functools.partial(
    pl.kernel,
    out_type=jax.ShapeDtypeStruct((2, _NP, _H), jnp.float32),
    mesh=_mesh,
    scratch_types=[
        [pltpu.VMEM((_K,), jnp.int32)] * 4,
        [pltpu.VMEM((_K,), jnp.int32)] * 4,
        [pltpu.VMEM((_K, _H), jnp.float32)] * 4,
        pltpu.VMEM_SHARED((_NP, _H), jnp.float32),
        pltpu.SemaphoreType.DMA((4,)),
        pltpu.SemaphoreType.DMA((4,)),
        pltpu.SemaphoreType.DMA((4,)),
        pltpu.SemaphoreType.DMA((2,)),
    ],
)
def _sc_scatter(g_hbm, row_hbm, col_hbm, zeros_hbm, out_hbm,
                rowv, colv, buf, acc, semR, semG, semC, semS):
    cid = lax.axis_index("c")
    sid = lax.axis_index("s")
    wid = sid * 2 + cid
    for t in range(_TPS // 128):
        pltpu.sync_copy(zeros_hbm, acc.at[pl.ds(sid * _TPS + t * 128, 128)])
    plsc.subcore_barrier()

    def rsl(j):
        return row_hbm.at[pl.ds(wid * _EW + j * _K, _K)]

    def csl(j):
        return col_hbm.at[pl.ds(wid * _EW + j * _K, _K)]

    def step(j, u, wait_scatter=True):
        s4, s2, a4 = u % 4, u % 2, (u + 2) % 4
        if wait_scatter:
            pltpu.make_async_copy(buf[a4], acc.at[colv[a4]],
                                  semS.at[s2]).wait()
        pltpu.make_async_copy(rsl(j + 2), rowv[a4], semR.at[a4]).wait()
        pltpu.async_copy(g_hbm.at[rowv[a4]], buf[a4], semG.at[a4])
        pltpu.async_copy(csl(j + 2), colv[a4], semC.at[a4])
        pltpu.make_async_copy(g_hbm.at[rowv[s4]], buf[s4], semG.at[s4]).wait()
        pltpu.make_async_copy(csl(j), colv[s4], semC.at[s4]).wait()
        pltpu.async_copy(buf[s4], acc.at[colv[s4]], semS.at[s2], add=True)
        pltpu.async_copy(rsl(j + 4), rowv[s4], semR.at[s4])

    for u in range(4):
        pltpu.async_copy(rsl(u), rowv[u], semR.at[u])
    for u in range(2):
        pltpu.async_copy(csl(u), colv[u], semC.at[u])
        pltpu.make_async_copy(rsl(u), rowv[u], semR.at[u]).wait()
        pltpu.async_copy(g_hbm.at[rowv[u]], buf[u], semG.at[u])
    for u in range(4):
        step(u, u, wait_scatter=(u >= 2))

    def body(i, carry):
        for u in range(4):
            step(4 * i + u, u)
        return carry

    lax.fori_loop(1, 31, body, 0)
    step(124, 0)
    pltpu.make_async_copy(buf[3], acc.at[colv[3]], semS.at[1]).wait()
    pltpu.make_async_copy(buf[0], acc.at[colv[0]], semS.at[0]).wait()
    for j, a in ((125, 1), (126, 2)):
        pltpu.make_async_copy(g_hbm.at[rowv[a]], buf[a], semG.at[a]).wait()
        pltpu.make_async_copy(csl(j), colv[a], semC.at[a]).wait()
    for j, a in ((127, 3), (128, 0)):
        pltpu.make_async_copy(rsl(j), rowv[a], semR.at[a]).wait()
    plsc.subcore_barrier()
    pltpu.sync_copy(acc.at[pl.ds(sid * _TPS, _TPS)],
                    out_hbm.at[cid, pl.ds(sid * _TPS, _TPS)])


def _tc_embed_body(x_ref, w_ref, b_ref, bng_ref, bnb_ref, deg_ref,
                   g_out, dis_out):
    h = jnp.dot(x_ref[...], w_ref[...], preferred_element_type=jnp.float32)
    h = jnp.maximum(h + b_ref[...], 0.0)
    scale = bng_ref[...] * np.float32(1.0 / np.sqrt(1.0 + 1e-5))
    h = h * scale + bnb_ref[...]
    dis = lax.rsqrt(deg_ref[...] + 1.0)
    dis_out[...] = dis
    g_out[...] = h * dis


def _tc_head_body(sp_ref, g_ref, dis_ref, sgw_ref, sgb_ref, w1_ref, b1_ref,
                  w2_ref, b2_ref, w3_ref, b3_ref, out_ref):
    s = sp_ref[0] + sp_ref[1] + g_ref[...]
    agg = s * dis_ref[...]
    h = jnp.dot(agg, sgw_ref[...], preferred_element_type=jnp.float32)
    h = jnp.maximum(h + sgb_ref[...], 0.0)
    h = jnp.dot(h, w1_ref[...], preferred_element_type=jnp.float32)
    h = jnp.maximum(h + b1_ref[...], 0.0)
    h = jnp.dot(h, w2_ref[...], preferred_element_type=jnp.float32)
    h = jnp.maximum(h + b2_ref[...], 0.0)
    z = jnp.dot(h, w3_ref[...], preferred_element_type=jnp.float32)
    z = z + b3_ref[...]
    out_ref[...] = 1.0 / (1.0 + jnp.exp(-z))


_B1 = 1000
_G1 = _N // _B1


def kernel(x, emb_w, emb_b, bn_g, bn_b, sg_w, sg_b, l1_w, l1_b,
           l2_w, l2_b, l3_w, l3_b, edge_index):
    ei = jnp.concatenate([edge_index, jnp.zeros((2, 512), jnp.int32)], axis=1)
    row = ei[0]
    col = ei[1]

    degp = _sc_degree(col,
                      jnp.zeros((_TPS, _DD), jnp.float32),
                      jnp.ones((_K, _DD), jnp.float32))
    deg = (degp[0, :_N, 0] + degp[1, :_N, 0]).reshape(_N, 1)

    g, dis = pl.pallas_call(
        _tc_embed_body,
        grid=(_G1,),
        in_specs=[
            pl.BlockSpec((_B1, _H), lambda i: (i, 0)),
            pl.BlockSpec((_H, _H), lambda i: (0, 0)),
            pl.BlockSpec((1, _H), lambda i: (0, 0)),
            pl.BlockSpec((1, _H), lambda i: (0, 0)),
            pl.BlockSpec((1, _H), lambda i: (0, 0)),
            pl.BlockSpec((_B1, 1), lambda i: (i, 0)),
        ],
        out_specs=[
            pl.BlockSpec((_B1, _H), lambda i: (i, 0)),
            pl.BlockSpec((_B1, 1), lambda i: (i, 0)),
        ],
        out_shape=[
            jax.ShapeDtypeStruct((_N, _H), jnp.float32),
            jax.ShapeDtypeStruct((_N, 1), jnp.float32),
        ],
    )(x, emb_w, emb_b.reshape(1, _H), bn_g.reshape(1, _H),
      bn_b.reshape(1, _H), deg)

    sp = _sc_scatter(g, row, col, jnp.zeros((128, _H), jnp.float32))
    sp = sp[:, :_N, :]

    out = pl.pallas_call(
        _tc_head_body,
        grid=(_G1,),
        in_specs=[
            pl.BlockSpec((2, _B1, _H), lambda i: (0, i, 0)),
            pl.BlockSpec((_B1, _H), lambda i: (i, 0)),
            pl.BlockSpec((_B1, 1), lambda i: (i, 0)),
            pl.BlockSpec((_H, _H), lambda i: (0, 0)),
            pl.BlockSpec((1, _H), lambda i: (0, 0)),
            pl.BlockSpec((_H, 64), lambda i: (0, 0)),
            pl.BlockSpec((1, 64), lambda i: (0, 0)),
            pl.BlockSpec((64, 32), lambda i: (0, 0)),
            pl.BlockSpec((1, 32), lambda i: (0, 0)),
            pl.BlockSpec((32, 18), lambda i: (0, 0)),
            pl.BlockSpec((1, 18), lambda i: (0, 0)),
        ],
        out_specs=pl.BlockSpec((_B1, 18), lambda i: (i, 0)),
        out_shape=jax.ShapeDtypeStruct((_N, 18), jnp.float32),
    )(sp, g, dis, sg_w, sg_b.reshape(1, _H), l1_w, l1_b.reshape(1, 64),
      l2_w, l2_b.reshape(1, 32), l3_w, l3_b.reshape(1, 18))
    return out


# speedup vs baseline: 31.7948x; 1.0374x over previous
import functools

import numpy as np
import jax
import jax.numpy as jnp
from jax import lax
from jax.experimental import pallas as pl
from jax.experimental.pallas import tpu as pltpu
from jax.experimental.pallas import tpu_sc as plsc

_N = 10000
_E = 320000
_H = 128
_NP = 10240
_NW = 32
_EW = _E // _NW
_K = 80
_CH = _EW // _K
_TPS = _NP // 16
_DD = 128

_mesh = plsc.VectorSubcoreMesh(core_axis_name="c", subcore_axis_name="s")


@functools.partial(
    pl.kernel,
    out_type=jax.ShapeDtypeStruct((2, _NP, _DD), jnp.float32),
    mesh=_mesh,
    scratch_types=[
        [pltpu.VMEM((_K,), jnp.int32)] * 8,
        pltpu.VMEM((_K, _DD), jnp.float32),
        pltpu.VMEM_SHARED((_NP, _DD), jnp.float32),
        pltpu.SemaphoreType.DMA((8,)),
        pltpu.SemaphoreType.DMA((4,)),
    ],
)
def _sc_degree(col_hbm, zeros_hbm, ones_hbm, out_hbm, colv,
               onesv, acc, semC, semS):
    cid = lax.axis_index("c")
    sid = lax.axis_index("s")
    wid = sid * 2 + cid
    pltpu.sync_copy(zeros_hbm, acc.at[pl.ds(sid * _TPS, _TPS)])
    pltpu.sync_copy(ones_hbm, onesv)
    plsc.subcore_barrier()

    def csl(j):
        return col_hbm.at[pl.ds(wid * _EW + j * _K, _K)]

    def step(j, u, wait_scatter=True):
        s8, s4, a8 = u % 8, u % 4, (u + 4) % 8
        if wait_scatter:
            pltpu.make_async_copy(onesv, acc.at[colv[a8]], semS.at[s4]).wait()
        pltpu.async_copy(csl(j + 4), colv[a8], semC.at[a8])
        pltpu.make_async_copy(csl(j), colv[s8], semC.at[s8]).wait()
        pltpu.async_copy(onesv, acc.at[colv[s8]], semS.at[s4], add=True)

    for u in range(4):
        pltpu.async_copy(csl(u), colv[u], semC.at[u])
    for u in range(8):
        step(u, u, wait_scatter=(u >= 4))

    def body(i, carry):
        for u in range(8):
            step(8 * i + u, u)
        return carry

    lax.fori_loop(1, 15, body, 0)
    for u in range(5):
        step(120 + u, u)
    for j, u in ((121, 1), (122, 2), (123, 3), (124, 0)):
        pltpu.make_async_copy(onesv, acc.at[colv[u % 8]], semS.at[u % 4]).wait()
    for j, a in ((125, 5), (126, 6), (127, 7), (128, 0)):
        pltpu.make_async_copy(csl(j), colv[a], semC.at[a]).wait()
    plsc.subcore_barrier()
    pltpu.sync_copy(acc.at[pl.ds(sid * _TPS, _TPS)],
                    out_hbm.at[cid, pl.ds(sid * _TPS, _TPS)])


@functools.partial(
    pl.kernel,
    out_type=jax.ShapeDtypeStruct((2, _NP, _H), jnp.float32),
    mesh=_mesh,
    scratch_types=[
        [pltpu.VMEM((_K,), jnp.int32)] * 4,
        [pltpu.VMEM((_K,), jnp.int32)] * 4,
        [pltpu.VMEM((_K, _H), jnp.float32)] * 4,
        pltpu.VMEM_SHARED((_NP, _H), jnp.float32),
        pltpu.SemaphoreType.DMA((4,)),
        pltpu.SemaphoreType.DMA((4,)),
        pltpu.SemaphoreType.DMA((4,)),
        pltpu.SemaphoreType.DMA((2,)),
    ],
)
def _sc_scatter(g_hbm, row_hbm, col_hbm, zeros_hbm, out_hbm,
                rowv, colv, buf, acc, semR, semG, semC, semS):
    cid = lax.axis_index("c")
    sid = lax.axis_index("s")
    wid = sid * 2 + cid
    for t in range(_TPS // 128):
        pltpu.sync_copy(zeros_hbm, acc.at[pl.ds(sid * _TPS + t * 128, 128)])
    plsc.subcore_barrier()

    def rsl(j):
        return row_hbm.at[pl.ds(wid * _EW + j * _K, _K)]

    def csl(j):
        return col_hbm.at[pl.ds(wid * _EW + j * _K, _K)]

    def step(j, u, wait_scatter=True):
        s4, s2, a4 = u % 4, u % 2, (u + 2) % 4
        if wait_scatter:
            pltpu.make_async_copy(buf[a4], acc.at[colv[a4]],
                                  semS.at[s2]).wait()
        pltpu.make_async_copy(rsl(j + 2), rowv[a4], semR.at[a4]).wait()
        pltpu.async_copy(g_hbm.at[rowv[a4]], buf[a4], semG.at[a4])
        pltpu.async_copy(csl(j + 2), colv[a4], semC.at[a4])
        pltpu.make_async_copy(g_hbm.at[rowv[s4]], buf[s4], semG.at[s4]).wait()
        pltpu.make_async_copy(csl(j), colv[s4], semC.at[s4]).wait()
        pltpu.async_copy(buf[s4], acc.at[colv[s4]], semS.at[s2], add=True)
        pltpu.async_copy(rsl(j + 4), rowv[s4], semR.at[s4])

    for u in range(4):
        pltpu.async_copy(rsl(u), rowv[u], semR.at[u])
    for u in range(2):
        pltpu.async_copy(csl(u), colv[u], semC.at[u])
        pltpu.make_async_copy(rsl(u), rowv[u], semR.at[u]).wait()
        pltpu.async_copy(g_hbm.at[rowv[u]], buf[u], semG.at[u])
    for u in range(4):
        step(u, u, wait_scatter=(u >= 2))

    def body(i, carry):
        for u in range(4):
            step(4 * i + u, u)
        return carry

    lax.fori_loop(1, 31, body, 0)
    step(124, 0)
    pltpu.make_async_copy(buf[3], acc.at[colv[3]], semS.at[1]).wait()
    pltpu.make_async_copy(buf[0], acc.at[colv[0]], semS.at[0]).wait()
    for j, a in ((125, 1), (126, 2)):
        pltpu.make_async_copy(g_hbm.at[rowv[a]], buf[a], semG.at[a]).wait()
        pltpu.make_async_copy(csl(j), colv[a], semC.at[a]).wait()
    for j, a in ((127, 3), (128, 0)):
        pltpu.make_async_copy(rsl(j), rowv[a], semR.at[a]).wait()
    plsc.subcore_barrier()
    pltpu.sync_copy(acc.at[pl.ds(sid * _TPS, _TPS)],
                    out_hbm.at[cid, pl.ds(sid * _TPS, _TPS)])


def _tc_embed_body(x_ref, w_ref, b_ref, bng_ref, bnb_ref, degp_ref,
                   g_out, dis_out):
    h = jnp.dot(x_ref[...], w_ref[...], preferred_element_type=jnp.float32)
    h = jnp.maximum(h + b_ref[...], 0.0)
    scale = bng_ref[...] * np.float32(1.0 / np.sqrt(1.0 + 1e-5))
    h = h * scale + bnb_ref[...]
    deg = degp_ref[0, :, 0:1] + degp_ref[1, :, 0:1]
    dis = lax.rsqrt(deg + 1.0)
    dis_out[...] = dis
    g_out[...] = h * dis


def _tc_head_body(sp_ref, g_ref, dis_ref, sgw_ref, sgb_ref, w1_ref, b1_ref,
                  w2_ref, b2_ref, w3_ref, b3_ref, out_ref):
    s = sp_ref[0] + sp_ref[1] + g_ref[...]
    agg = s * dis_ref[...]
    h = jnp.dot(agg, sgw_ref[...], preferred_element_type=jnp.float32)
    h = jnp.maximum(h + sgb_ref[...], 0.0)
    h = jnp.dot(h, w1_ref[...], preferred_element_type=jnp.float32)
    h = jnp.maximum(h + b1_ref[...], 0.0)
    h = jnp.dot(h, w2_ref[...], preferred_element_type=jnp.float32)
    h = jnp.maximum(h + b2_ref[...], 0.0)
    z = jnp.dot(h, w3_ref[...], preferred_element_type=jnp.float32)
    z = z + b3_ref[...]
    out_ref[...] = 1.0 / (1.0 + jnp.exp(-z))


_B1 = 1000
_G1 = _N // _B1


def kernel(x, emb_w, emb_b, bn_g, bn_b, sg_w, sg_b, l1_w, l1_b,
           l2_w, l2_b, l3_w, l3_b, edge_index):
    ei = jnp.concatenate([edge_index, jnp.zeros((2, 512), jnp.int32)], axis=1)
    row = ei[0]
    col = ei[1]

    degp = _sc_degree(col,
                      jnp.zeros((_TPS, _DD), jnp.float32),
                      jnp.ones((_K, _DD), jnp.float32))

    g, dis = pl.pallas_call(
        _tc_embed_body,
        grid=(_G1,),
        in_specs=[
            pl.BlockSpec((_B1, _H), lambda i: (i, 0)),
            pl.BlockSpec((_H, _H), lambda i: (0, 0)),
            pl.BlockSpec((1, _H), lambda i: (0, 0)),
            pl.BlockSpec((1, _H), lambda i: (0, 0)),
            pl.BlockSpec((1, _H), lambda i: (0, 0)),
            pl.BlockSpec((2, _B1, _DD), lambda i: (0, i, 0)),
        ],
        out_specs=[
            pl.BlockSpec((_B1, _H), lambda i: (i, 0)),
            pl.BlockSpec((_B1, 1), lambda i: (i, 0)),
        ],
        out_shape=[
            jax.ShapeDtypeStruct((_N, _H), jnp.float32),
            jax.ShapeDtypeStruct((_N, 1), jnp.float32),
        ],
    )(x, emb_w, emb_b.reshape(1, _H), bn_g.reshape(1, _H),
      bn_b.reshape(1, _H), degp)

    sp = _sc_scatter(g, row, col, jnp.zeros((128, _H), jnp.float32))

    out = pl.pallas_call(
        _tc_head_body,
        grid=(_G1,),
        in_specs=[
            pl.BlockSpec((2, _B1, _H), lambda i: (0, i, 0)),
            pl.BlockSpec((_B1, _H), lambda i: (i, 0)),
            pl.BlockSpec((_B1, 1), lambda i: (i, 0)),
            pl.BlockSpec((_H, _H), lambda i: (0, 0)),
            pl.BlockSpec((1, _H), lambda i: (0, 0)),
            pl.BlockSpec((_H, 64), lambda i: (0, 0)),
            pl.BlockSpec((1, 64), lambda i: (0, 0)),
            pl.BlockSpec((64, 32), lambda i: (0, 0)),
            pl.BlockSpec((1, 32), lambda i: (0, 0)),
            pl.BlockSpec((32, 18), lambda i: (0, 0)),
            pl.BlockSpec((1, 18), lambda i: (0, 0)),
        ],
        out_specs=pl.BlockSpec((_B1, 18), lambda i: (i, 0)),
        out_shape=jax.ShapeDtypeStruct((_N, 18), jnp.float32),
    )(sp, g, dis, sg_w, sg_b.reshape(1, _H), l1_w, l1_b.reshape(1, 64),
      l2_w, l2_b.reshape(1, 32), l3_w, l3_b.reshape(1, 18))
    return out


# speedup vs baseline: 32.0589x; 1.0083x over previous
import functools

import numpy as np
import jax
import jax.numpy as jnp
from jax import lax
from jax.experimental import pallas as pl
from jax.experimental.pallas import tpu as pltpu
from jax.experimental.pallas import tpu_sc as plsc

_N = 10000
_E = 320000
_H = 128
_NP = 10240
_NW = 32
_EW = _E // _NW
_K = 80
_CH = _EW // _K
_TPS = _NP // 16
_DD = 128

_mesh = plsc.VectorSubcoreMesh(core_axis_name="c", subcore_axis_name="s")


@functools.partial(
    pl.kernel,
    out_type=jax.ShapeDtypeStruct((2, _NP, _DD), jnp.float32),
    mesh=_mesh,
    scratch_types=[
        [pltpu.VMEM((_K,), jnp.int32)] * 8,
        pltpu.VMEM((_K, _DD), jnp.float32),
        pltpu.VMEM_SHARED((_NP, _DD), jnp.float32),
        pltpu.SemaphoreType.DMA((8,)),
        pltpu.SemaphoreType.DMA((4,)),
    ],
)
def _sc_degree(col_hbm, zeros_hbm, ones_hbm, out_hbm, colv,
               onesv, acc, semC, semS):
    cid = lax.axis_index("c")
    sid = lax.axis_index("s")
    wid = sid * 2 + cid
    pltpu.sync_copy(zeros_hbm, acc.at[pl.ds(sid * _TPS, _TPS)])
    pltpu.sync_copy(ones_hbm, onesv)
    plsc.subcore_barrier()

    def csl(j):
        jq = jnp.minimum(j, _CH - 1)
        return col_hbm.at[pl.ds(wid * _EW + jq * _K, _K)]

    def step(j, u, wait_scatter=True):
        s8, s4, a8 = u % 8, u % 4, (u + 4) % 8
        if wait_scatter:
            pltpu.make_async_copy(onesv, acc.at[colv[a8]], semS.at[s4]).wait()
        pltpu.async_copy(csl(j + 4), colv[a8], semC.at[a8])
        pltpu.make_async_copy(csl(j), colv[s8], semC.at[s8]).wait()
        pltpu.async_copy(onesv, acc.at[colv[s8]], semS.at[s4], add=True)

    for u in range(4):
        pltpu.async_copy(csl(u), colv[u], semC.at[u])
    for u in range(8):
        step(u, u, wait_scatter=(u >= 4))

    def body(i, carry):
        for u in range(8):
            step(8 * i + u, u)
        return carry

    lax.fori_loop(1, 15, body, 0)
    for u in range(5):
        step(120 + u, u)
    for j, u in ((121, 1), (122, 2), (123, 3), (124, 0)):
        pltpu.make_async_copy(onesv, acc.at[colv[u % 8]], semS.at[u % 4]).wait()
    for j, a in ((125, 5), (126, 6), (127, 7), (128, 0)):
        pltpu.make_async_copy(csl(j), colv[a], semC.at[a]).wait()
    plsc.subcore_barrier()
    pltpu.sync_copy(acc.at[pl.ds(sid * _TPS, _TPS)],
                    out_hbm.at[cid, pl.ds(sid * _TPS, _TPS)])


@functools.partial(
    pl.kernel,
    out_type=jax.ShapeDtypeStruct((2, _NP, _H), jnp.float32),
    mesh=_mesh,
    scratch_types=[
        [pltpu.VMEM((_K,), jnp.int32)] * 4,
        [pltpu.VMEM((_K,), jnp.int32)] * 4,
        [pltpu.VMEM((_K, _H), jnp.float32)] * 4,
        pltpu.VMEM_SHARED((_NP, _H), jnp.float32),
        pltpu.SemaphoreType.DMA((4,)),
        pltpu.SemaphoreType.DMA((4,)),
        pltpu.SemaphoreType.DMA((4,)),
        pltpu.SemaphoreType.DMA((2,)),
    ],
)
def _sc_scatter(g_hbm, row_hbm, col_hbm, zeros_hbm, out_hbm,
                rowv, colv, buf, acc, semR, semG, semC, semS):
    cid = lax.axis_index("c")
    sid = lax.axis_index("s")
    wid = sid * 2 + cid
    for t in range(_TPS // 128):
        pltpu.sync_copy(zeros_hbm, acc.at[pl.ds(sid * _TPS + t * 128, 128)])
    plsc.subcore_barrier()

    def rsl(j):
        jq = jnp.minimum(j, _CH - 1)
        return row_hbm.at[pl.ds(wid * _EW + jq * _K, _K)]

    def csl(j):
        jq = jnp.minimum(j, _CH - 1)
        return col_hbm.at[pl.ds(wid * _EW + jq * _K, _K)]

    def step(j, u, wait_scatter=True):
        s4, s2, a4 = u % 4, u % 2, (u + 2) % 4
        if wait_scatter:
            pltpu.make_async_copy(buf[a4], acc.at[colv[a4]],
                                  semS.at[s2]).wait()
        pltpu.make_async_copy(rsl(j + 2), rowv[a4], semR.at[a4]).wait()
        pltpu.async_copy(g_hbm.at[rowv[a4]], buf[a4], semG.at[a4])
        pltpu.async_copy(csl(j + 2), colv[a4], semC.at[a4])
        pltpu.make_async_copy(g_hbm.at[rowv[s4]], buf[s4], semG.at[s4]).wait()
        pltpu.make_async_copy(csl(j), colv[s4], semC.at[s4]).wait()
        pltpu.async_copy(buf[s4], acc.at[colv[s4]], semS.at[s2], add=True)
        pltpu.async_copy(rsl(j + 4), rowv[s4], semR.at[s4])

    for u in range(4):
        pltpu.async_copy(rsl(u), rowv[u], semR.at[u])
    for u in range(2):
        pltpu.async_copy(csl(u), colv[u], semC.at[u])
        pltpu.make_async_copy(rsl(u), rowv[u], semR.at[u]).wait()
        pltpu.async_copy(g_hbm.at[rowv[u]], buf[u], semG.at[u])
    for u in range(4):
        step(u, u, wait_scatter=(u >= 2))

    def body(i, carry):
        for u in range(4):
            step(4 * i + u, u)
        return carry

    lax.fori_loop(1, 31, body, 0)
    step(124, 0)
    pltpu.make_async_copy(buf[3], acc.at[colv[3]], semS.at[1]).wait()
    pltpu.make_async_copy(buf[0], acc.at[colv[0]], semS.at[0]).wait()
    for j, a in ((125, 1), (126, 2)):
        pltpu.make_async_copy(g_hbm.at[rowv[a]], buf[a], semG.at[a]).wait()
        pltpu.make_async_copy(csl(j), colv[a], semC.at[a]).wait()
    for j, a in ((127, 3), (128, 0)):
        pltpu.make_async_copy(rsl(j), rowv[a], semR.at[a]).wait()
    plsc.subcore_barrier()
    pltpu.sync_copy(acc.at[pl.ds(sid * _TPS, _TPS)],
                    out_hbm.at[cid, pl.ds(sid * _TPS, _TPS)])


def _tc_embed_body(x_ref, w_ref, b_ref, bng_ref, bnb_ref, degp_ref,
                   g_out, dis_out):
    h = jnp.dot(x_ref[...], w_ref[...], preferred_element_type=jnp.float32)
    h = jnp.maximum(h + b_ref[...], 0.0)
    scale = bng_ref[...] * np.float32(1.0 / np.sqrt(1.0 + 1e-5))
    h = h * scale + bnb_ref[...]
    deg = degp_ref[0, :, 0:1] + degp_ref[1, :, 0:1]
    dis = lax.rsqrt(deg + 1.0)
    dis_out[...] = dis
    g_out[...] = h * dis


def _tc_head_body(sp_ref, g_ref, dis_ref, sgw_ref, sgb_ref, w1_ref, b1_ref,
                  w2_ref, b2_ref, w3_ref, b3_ref, out_ref):
    s = sp_ref[0] + sp_ref[1] + g_ref[...]
    agg = s * dis_ref[...]
    h = jnp.dot(agg, sgw_ref[...], preferred_element_type=jnp.float32)
    h = jnp.maximum(h + sgb_ref[...], 0.0)
    h = jnp.dot(h, w1_ref[...], preferred_element_type=jnp.float32)
    h = jnp.maximum(h + b1_ref[...], 0.0)
    h = jnp.dot(h, w2_ref[...], preferred_element_type=jnp.float32)
    h = jnp.maximum(h + b2_ref[...], 0.0)
    z = jnp.dot(h, w3_ref[...], preferred_element_type=jnp.float32)
    z = z + b3_ref[...]
    out_ref[...] = 1.0 / (1.0 + jnp.exp(-z))


_B1 = 2000
_G1 = _N // _B1


def kernel(x, emb_w, emb_b, bn_g, bn_b, sg_w, sg_b, l1_w, l1_b,
           l2_w, l2_b, l3_w, l3_b, edge_index):
    row = edge_index[0]
    col = edge_index[1]

    degp = _sc_degree(col,
                      jnp.zeros((_TPS, _DD), jnp.float32),
                      jnp.ones((_K, _DD), jnp.float32))

    g, dis = pl.pallas_call(
        _tc_embed_body,
        grid=(_G1,),
        in_specs=[
            pl.BlockSpec((_B1, _H), lambda i: (i, 0)),
            pl.BlockSpec((_H, _H), lambda i: (0, 0)),
            pl.BlockSpec((1, _H), lambda i: (0, 0)),
            pl.BlockSpec((1, _H), lambda i: (0, 0)),
            pl.BlockSpec((1, _H), lambda i: (0, 0)),
            pl.BlockSpec((2, _B1, _DD), lambda i: (0, i, 0)),
        ],
        out_specs=[
            pl.BlockSpec((_B1, _H), lambda i: (i, 0)),
            pl.BlockSpec((_B1, 1), lambda i: (i, 0)),
        ],
        out_shape=[
            jax.ShapeDtypeStruct((_N, _H), jnp.float32),
            jax.ShapeDtypeStruct((_N, 1), jnp.float32),
        ],
    )(x, emb_w, emb_b.reshape(1, _H), bn_g.reshape(1, _H),
      bn_b.reshape(1, _H), degp)

    sp = _sc_scatter(g, row, col, jnp.zeros((128, _H), jnp.float32))

    out = pl.pallas_call(
        _tc_head_body,
        grid=(_G1,),
        in_specs=[
            pl.BlockSpec((2, _B1, _H), lambda i: (0, i, 0)),
            pl.BlockSpec((_B1, _H), lambda i: (i, 0)),
            pl.BlockSpec((_B1, 1), lambda i: (i, 0)),
            pl.BlockSpec((_H, _H), lambda i: (0, 0)),
            pl.BlockSpec((1, _H), lambda i: (0, 0)),
            pl.BlockSpec((_H, 64), lambda i: (0, 0)),
            pl.BlockSpec((1, 64), lambda i: (0, 0)),
            pl.BlockSpec((64, 32), lambda i: (0, 0)),
            pl.BlockSpec((1, 32), lambda i: (0, 0)),
            pl.BlockSpec((32, 18), lambda i: (0, 0)),
            pl.BlockSpec((1, 18), lambda i: (0, 0)),
        ],
        out_specs=pl.BlockSpec((_B1, 18), lambda i: (i, 0)),
        out_shape=jax.ShapeDtypeStruct((_N, 18), jnp.float32),
    )(sp, g, dis, sg_w, sg_b.reshape(1, _H), l1_w, l1_b.reshape(1, 64),
      l2_w, l2_b.reshape(1, 32), l3_w, l3_b.reshape(1, 18))
    return out
